# baseline (device time: 28078 ns/iter reference)
import jax
import jax.numpy as jnp
from jax import lax
from jax.experimental import pallas as pl
from jax.experimental.pallas import tpu as pltpu

N_DEV = 4
E_PER = 2
CAP_E = 96
CAP = E_PER * CAP_E


def kernel(x, assign, W1, W2):
    t, d = x.shape
    e_per, _, f = W1.shape
    assert e_per == E_PER
    a_row = assign.reshape(1, t)
    a_col = assign.reshape(t, 1)

    def body(x_ref, a_ref, ac_ref, w1_ref, w2_ref, out_ref,
             xsend, psend, xbuf, rbuf,
             xs_sems, rs_sems, xr_sems, rr_sems):
        me = lax.axis_index("i")

        barrier = pltpu.get_barrier_semaphore()
        for j in range(1, N_DEV):
            p = lax.rem(me + j, N_DEV)
            pl.semaphore_signal(barrier, inc=1, device_id=(p,),
                                device_id_type=pl.DeviceIdType.MESH)
        pl.semaphore_wait(barrier, N_DEV - 1)

        xb = x_ref[...].astype(jnp.bfloat16)
        n_exp = E_PER * N_DEV
        brow = lax.rem(a_ref[...] + (n_exp - E_PER * me), n_exp)
        bcol = lax.rem(ac_ref[...] + (n_exp - E_PER * me), n_exp)

        ii = lax.broadcasted_iota(jnp.int32, (t, t), 0)
        jj = lax.broadcasted_iota(jnp.int32, (t, t), 1)
        U = (ii < jj).astype(jnp.bfloat16)
        L = (ii > jj).astype(jnp.bfloat16)
        iota_c = lax.broadcasted_iota(jnp.int32, (CAP, 1), 0)
        iota_r = lax.broadcasted_iota(jnp.int32, (1, CAP), 1)

        B8 = brow == lax.broadcasted_iota(jnp.int32, (n_exp, 1), 0)
        CS8 = jnp.dot(B8.astype(jnp.bfloat16), U,
                      preferred_element_type=jnp.float32
                      ).astype(jnp.int32)
        B8c = bcol == lax.broadcasted_iota(jnp.int32, (1, n_exp), 1)
        CS8c = jnp.dot(L, B8c.astype(jnp.bfloat16),
                       preferred_element_type=jnp.float32
                       ).astype(jnp.int32)

        def perm_pair(j):
            n0 = E_PER * j
            P = (((CS8[n0:n0 + 1, :] == iota_c) & B8[n0:n0 + 1, :])
                 | ((CS8[n0 + 1:n0 + 2, :] == iota_c - CAP_E)
                    & B8[n0 + 1:n0 + 2, :])
                 ).astype(jnp.bfloat16)
            Pt = (((CS8c[:, n0:n0 + 1] == iota_r) & B8c[:, n0:n0 + 1])
                  | ((CS8c[:, n0 + 1:n0 + 2] == iota_r - CAP_E)
                     & B8c[:, n0 + 1:n0 + 2])
                  ).astype(jnp.bfloat16)
            return P, Pt

        Pt_all = [None] * N_DEV
        xc_own = None
        sends = []
        for j in (1, 2, 3, 0):
            P, Pt = perm_pair(j)
            Pt_all[j] = Pt
            xc = jnp.dot(P, xb, preferred_element_type=jnp.float32
                         ).astype(jnp.bfloat16)
            if j == 0:
                xc_own = xc
                continue
            slot = N_DEV - j - 1
            xsend[j - 1] = xc
            rx = pltpu.make_async_remote_copy(
                src_ref=xsend.at[j - 1], dst_ref=xbuf.at[slot],
                send_sem=xs_sems.at[j - 1], recv_sem=xr_sems.at[slot],
                device_id=(lax.rem(me + j, N_DEV),),
                device_id_type=pl.DeviceIdType.MESH)
            rx.start()
            sends.append(rx)

        W1cat = jnp.concatenate(
            [w1_ref[0].astype(jnp.bfloat16), w1_ref[1].astype(jnp.bfloat16)],
            axis=1)
        W2cat = w2_ref[...].astype(jnp.bfloat16).reshape(E_PER * f, d)
        hr = lax.broadcasted_iota(jnp.int32, (CAP, 1), 0) // CAP_E
        hc = lax.broadcasted_iota(jnp.int32, (1, E_PER * f), 1) // f
        HM = hr == hc

        def apply_group(xg):
            h = jnp.dot(xg, W1cat, preferred_element_type=jnp.float32)
            hb = jnp.maximum(h, 0.0).astype(jnp.bfloat16)
            hm = jnp.where(HM, hb, jnp.zeros_like(hb))
            return jnp.dot(hm, W2cat, preferred_element_type=jnp.float32)

        y_own = apply_group(xc_own).astype(jnp.bfloat16)

        for i in range(N_DEV - 1):
            pltpu.make_async_remote_copy(
                src_ref=xsend.at[0], dst_ref=xbuf.at[i],
                send_sem=xs_sems.at[0], recv_sem=xr_sems.at[i],
                device_id=(me,), device_id_type=pl.DeviceIdType.MESH,
            ).wait_recv()
            psend[i] = apply_group(xbuf[i]).astype(jnp.bfloat16)
            owner = lax.rem(me + i + 1, N_DEV)
            rr = pltpu.make_async_remote_copy(
                src_ref=psend.at[i], dst_ref=rbuf.at[2 - i],
                send_sem=rs_sems.at[i], recv_sem=rr_sems.at[2 - i],
                device_id=(owner,), device_id_type=pl.DeviceIdType.MESH)
            rr.start()
            sends.append(rr)

        out = jnp.dot(Pt_all[0], y_own, preferred_element_type=jnp.float32)
        for k in range(N_DEV - 1):
            pltpu.make_async_remote_copy(
                src_ref=psend.at[0], dst_ref=rbuf.at[k],
                send_sem=rs_sems.at[0], recv_sem=rr_sems.at[k],
                device_id=(me,), device_id_type=pl.DeviceIdType.MESH,
            ).wait_recv()
            out = out + jnp.dot(Pt_all[k + 1], rbuf[k],
                                preferred_element_type=jnp.float32)
        out_ref[...] = out

        for dsc in sends:
            dsc.wait_send()

    return pl.pallas_call(
        body,
        out_shape=jax.ShapeDtypeStruct((t, d), jnp.float32),
        in_specs=[pl.BlockSpec(memory_space=pltpu.VMEM)] * 5,
        out_specs=pl.BlockSpec(memory_space=pltpu.VMEM),
        scratch_shapes=[
            pltpu.VMEM((N_DEV - 1, CAP, d), jnp.bfloat16),
            pltpu.VMEM((N_DEV - 1, CAP, d), jnp.bfloat16),
            pltpu.VMEM((N_DEV - 1, CAP, d), jnp.bfloat16),
            pltpu.VMEM((N_DEV - 1, CAP, d), jnp.bfloat16),
            pltpu.SemaphoreType.DMA((N_DEV - 1,)),
            pltpu.SemaphoreType.DMA((N_DEV - 1,)),
            pltpu.SemaphoreType.DMA((N_DEV - 1,)),
            pltpu.SemaphoreType.DMA((N_DEV - 1,)),
        ],
        compiler_params=pltpu.CompilerParams(collective_id=0),
    )(x, a_row, a_col, W1, W2)
